# Initial kernel scaffold; baseline (speedup 1.0000x reference)
import jax
import jax.numpy as jnp
from jax.experimental import pallas as pl


def kernel(features, edge_index, W_self, W_neigh, b):
    raise NotImplementedError("write your pallas kernel here")


# speedup vs baseline: 6.2163x; 6.2163x over previous
import jax
import jax.numpy as jnp
from jax import lax
from jax.experimental import pallas as pl
from jax.experimental.pallas import tpu as pltpu
from jax.experimental.pallas import tpu_sc as plsc

N = 10000
NPAD = 10240
D = 128
DH = D // 2
E = 320000
NC = 2
NS = 16
EPW = E // NS
CH = 80
NCHUNK = EPW // CH
RPT = NPAD // NS
RCH = 128
NRC = RPT // RCH
DPAD = 10240
DPT = DPAD // NS


def _sc_body(feat_a, feat_b, srcr, dstr, acc_out, deg_out,
             src_v, dst_v, rows_v, zbuf, zdeg, ones_v, acc_sh, deg_sh, sem):
    c = lax.axis_index("c")
    s = lax.axis_index("s")

    zv = jnp.zeros((16,), jnp.float32)

    def zb_row(r, carry):
        for k in range(DH // 16):
            zbuf[r, pl.ds(k * 16, 16)] = zv
        return carry

    lax.fori_loop(0, RCH, zb_row, 0)

    def zd_row(i, carry):
        zdeg[pl.ds(i * 16, 16)] = zv
        return carry

    lax.fori_loop(0, DPT // 16, zd_row, 0)

    ov = jnp.ones((16,), jnp.float32)
    for k in range(CH // 16):
        ones_v[pl.ds(k * 16, 16)] = ov

    for k in range(NRC):
        pltpu.sync_copy(zbuf, acc_sh.at[pl.ds(s * RPT + k * RCH, RCH)])
    pltpu.sync_copy(zdeg, deg_sh.at[pl.ds(s * DPT, DPT)])
    plsc.subcore_barrier()

    pltpu.sync_copy(srcr.at[s], src_v)
    pltpu.sync_copy(dstr.at[s], dst_v)

    def edge_loop(feat):
        def step(j, carry):
            pltpu.async_copy(feat.at[src_v.at[j]], rows_v, sem).wait()
            pltpu.sync_copy(rows_v, acc_sh.at[dst_v.at[j]], add=True)
            return carry

        lax.fori_loop(0, NCHUNK, step, 0)

    @pl.when(c == 0)
    def _():
        edge_loop(feat_a)

        def dstep(j, carry):
            pltpu.sync_copy(ones_v, deg_sh.at[dst_v.at[j]], add=True)
            return carry

        lax.fori_loop(0, NCHUNK, dstep, 0)

    @pl.when(c == 1)
    def _():
        edge_loop(feat_b)

    plsc.subcore_barrier()

    for k in range(NRC):
        r0 = s * RPT + k * RCH
        pltpu.sync_copy(acc_sh.at[pl.ds(r0, RCH)], zbuf)
        pltpu.sync_copy(zbuf, acc_out.at[c, pl.ds(r0, RCH)])

    @pl.when(c == 0)
    def _():
        pltpu.sync_copy(deg_sh.at[pl.ds(s * DPT, DPT)], zdeg)
        pltpu.sync_copy(zdeg, deg_out.at[pl.ds(s * DPT, DPT)])


_sc_agg = pl.kernel(
    _sc_body,
    out_type=(jax.ShapeDtypeStruct((NC, NPAD, DH), jnp.float32),
              jax.ShapeDtypeStruct((DPAD,), jnp.float32)),
    mesh=plsc.VectorSubcoreMesh(core_axis_name="c", subcore_axis_name="s"),
    scratch_types=[
        pltpu.VMEM((NCHUNK, CH), jnp.int32),
        pltpu.VMEM((NCHUNK, CH), jnp.int32),
        pltpu.VMEM((CH, DH), jnp.float32),
        pltpu.VMEM((RCH, DH), jnp.float32),
        pltpu.VMEM((DPT,), jnp.float32),
        pltpu.VMEM((CH,), jnp.float32),
        pltpu.VMEM_SHARED((NPAD, DH), jnp.float32),
        pltpu.VMEM_SHARED((DPAD,), jnp.float32),
        pltpu.SemaphoreType.DMA,
    ],
    compiler_params=pltpu.CompilerParams(use_tc_tiling_on_sc=False),
)


def _tc_body(x_ref, p_ref, d_ref, ws_ref, wna_ref, wnb_ref, b_ref, o_ref):
    x = x_ref[...]
    scale = 1.0 / jnp.maximum(d_ref[...], 1.0)
    ha = p_ref[0, :N] * scale
    hb = p_ref[1, :N] * scale
    dims = (((1,), (1,)), ((), ()))
    o_ref[...] = (
        lax.dot_general(x, ws_ref[...], dims,
                        preferred_element_type=jnp.float32)
        + lax.dot_general(ha, wna_ref[...], dims,
                          preferred_element_type=jnp.float32)
        + lax.dot_general(hb, wnb_ref[...], dims,
                          preferred_element_type=jnp.float32)
        + b_ref[...]
    )


_tc_combine = pl.pallas_call(
    _tc_body,
    out_shape=jax.ShapeDtypeStruct((N, D), jnp.float32),
)


def kernel(features, edge_index, W_self, W_neigh, b):
    src = edge_index[0].astype(jnp.int32).reshape(NS, NCHUNK, CH)
    dst = edge_index[1].astype(jnp.int32).reshape(NS, NCHUNK, CH)
    feat_a = features[:, :DH]
    feat_b = features[:, DH:]
    acc, deg = _sc_agg(feat_a, feat_b, src, dst)
    return _tc_combine(features, acc, deg[:N].reshape(N, 1), W_self,
                       W_neigh[:, :DH], W_neigh[:, DH:], b.reshape(1, D))
